# baseline (speedup 1.0000x reference)
import jax
import jax.numpy as jnp
from jax import lax
from jax.experimental import pallas as pl
from jax.experimental.pallas import tpu as pltpu
from jax.experimental.pallas import tpu_sc as plsc

ROWS = 64
COLS = 32768
LANES = 16
CHUNKS = COLS // LANES
NWORKERS = 32
ROWS_PER_WORKER = ROWS // NWORKERS

_NEG_INF = float("-inf")
_BIG_I32 = 2**30


def _argmax_pass(row_ref, lane_iota):
  def body(c, carry):
    best, bestc = carry
    v = row_ref[pl.ds(c * LANES, LANES)]
    m = v > best
    best = jnp.where(m, v, best)
    bestc = jnp.where(m, c, bestc)
    return best, bestc

  init = (jnp.full((LANES,), _NEG_INF, jnp.float32),
          jnp.zeros((LANES,), jnp.int32))
  best, bestc = lax.fori_loop(0, CHUNKS, body, init)
  idx = bestc * LANES + lane_iota
  maxv = jnp.max(best)
  cand = jnp.where(best == maxv, idx, _BIG_I32)
  return jnp.min(cand)


def _top3_row(row_ref, lane_iota):
  i1 = _argmax_pass(row_ref, lane_iota)
  results = [i1]
  for _ in range(2):
    prev = results[-1]
    c_prev = prev // LANES
    l_prev = prev - c_prev * LANES
    chunk = row_ref[pl.ds(c_prev * LANES, LANES)]
    row_ref[pl.ds(c_prev * LANES, LANES)] = jnp.where(
        lane_iota == l_prev, _NEG_INF, chunk)
    results.append(_argmax_pass(row_ref, lane_iota))
  return results


def _sc_kernel(x_hbm, out_hbm, buf0, buf1, outbuf, sem0, sem1):
  wid = lax.axis_index("c") * 16 + lax.axis_index("s")
  r0 = wid * ROWS_PER_WORKER
  lane_iota = lax.broadcasted_iota(jnp.int32, (LANES,), 0)

  cp0 = pltpu.async_copy(x_hbm.at[r0], buf0, sem0)
  cp1 = pltpu.async_copy(x_hbm.at[r0 + 1], buf1, sem1)

  cp0.wait()
  a1, a2, a3 = _top3_row(buf0, lane_iota)
  cp1.wait()
  b1, b2, b3 = _top3_row(buf1, lane_iota)

  vals = [a1, a2, a3, b1, b2, b3]
  res = jnp.zeros((LANES,), jnp.float32)
  for lane, v in enumerate(vals):
    res = jnp.where(lane_iota == lane, v.astype(jnp.float32), res)
  outbuf[...] = res
  pltpu.sync_copy(outbuf, out_hbm.at[wid])


@jax.jit
def kernel(x):
  mesh = plsc.VectorSubcoreMesh(core_axis_name="c", subcore_axis_name="s")
  k = pl.kernel(
      _sc_kernel,
      out_type=jax.ShapeDtypeStruct((NWORKERS, LANES), jnp.float32),
      mesh=mesh,
      compiler_params=pltpu.CompilerParams(needs_layout_passes=False),
      scratch_types=[
          pltpu.VMEM((COLS,), jnp.float32),
          pltpu.VMEM((COLS,), jnp.float32),
          pltpu.VMEM((LANES,), jnp.float32),
          pltpu.SemaphoreType.DMA,
          pltpu.SemaphoreType.DMA,
      ],
  )
  staged = k(x)
  return staged[:, :6].reshape(ROWS, 3)


# speedup vs baseline: 2.6388x; 2.6388x over previous
import jax
import jax.numpy as jnp
from jax import lax
from jax.experimental import pallas as pl
from jax.experimental.pallas import tpu as pltpu
from jax.experimental.pallas import tpu_sc as plsc

ROWS = 64
COLS = 32768
LANES = 16
NWORKERS = 32
ROWS_PER_WORKER = ROWS // NWORKERS

NBLK = 16
BCHUNKS = COLS // (NBLK * LANES)

_NEG_INF = float("-inf")
_BIG_I32 = 2**30


def _block_maxima(row_ref, lane_iota):
  ninf = jnp.full((LANES,), _NEG_INF, jnp.float32)

  def blk_body(j, bvec):
    base = j * (BCHUNKS * LANES)

    def body(c, accs):
      a0, a1, a2, a3 = accs
      o = base + c * (16 * LANES)
      for u in range(0, 16, 4):
        a0 = jnp.maximum(a0, row_ref[pl.ds(o + (u + 0) * LANES, LANES)])
        a1 = jnp.maximum(a1, row_ref[pl.ds(o + (u + 1) * LANES, LANES)])
        a2 = jnp.maximum(a2, row_ref[pl.ds(o + (u + 2) * LANES, LANES)])
        a3 = jnp.maximum(a3, row_ref[pl.ds(o + (u + 3) * LANES, LANES)])
      return a0, a1, a2, a3

    a0, a1, a2, a3 = lax.fori_loop(0, BCHUNKS // 16, body,
                                   (ninf, ninf, ninf, ninf))
    acc = jnp.maximum(jnp.maximum(a0, a1), jnp.maximum(a2, a3))
    bm = jnp.max(acc)
    return jnp.where(lane_iota == j, bm, bvec)

  return lax.fori_loop(0, NBLK, blk_body, ninf)


def _top3_block_ids(bvec, lane_iota):
  ids = []
  b = bvec
  for _ in range(3):
    m = jnp.max(b)
    j = jnp.min(jnp.where(b == m, lane_iota, _BIG_I32))
    ids.append(j)
    b = jnp.where(lane_iota == j, _NEG_INF, b)
  a, bb, c = ids
  lo = jnp.minimum(jnp.minimum(a, bb), c)
  hi = jnp.maximum(jnp.maximum(a, bb), c)
  mid = a + bb + c - lo - hi
  return lo, mid, hi


def _argmax_blocks(row_ref, lane_iota, block_ids):
  ninf = jnp.full((LANES,), _NEG_INF, jnp.float32)
  zero = jnp.zeros((LANES,), jnp.int32)
  carry = (ninf, zero, ninf, zero, ninf, zero, ninf, zero)

  for j in block_ids:
    cbase = j * BCHUNKS

    def body(c, accs, cbase=cbase):
      b0, c0, b1, c1, b2, c2, b3, c3 = accs
      cc = cbase + c * 8
      o = cc * LANES
      bs = [b0, b1, b2, b3]
      cs = [c0, c1, c2, c3]
      for u in range(8):
        k = u % 4
        v = row_ref[pl.ds(o + u * LANES, LANES)]
        m = v > bs[k]
        bs[k] = jnp.where(m, v, bs[k])
        cs[k] = jnp.where(m, cc + u, cs[k])
      return (bs[0], cs[0], bs[1], cs[1], bs[2], cs[2], bs[3], cs[3])

    carry = lax.fori_loop(0, BCHUNKS // 8, body, carry)

  def merge(bv_a, cv_a, bv_b, cv_b):
    take = (bv_b > bv_a) | ((bv_b == bv_a) & (cv_b < cv_a))
    return jnp.where(take, bv_b, bv_a), jnp.where(take, cv_b, cv_a)

  b0, c0, b1, c1, b2, c2, b3, c3 = carry
  ba, ca = merge(b0, c0, b1, c1)
  bb, cb = merge(b2, c2, b3, c3)
  best, bestc = merge(ba, ca, bb, cb)

  idx = bestc * LANES + lane_iota
  maxv = jnp.max(best)
  return jnp.min(jnp.where(best == maxv, idx, _BIG_I32))


def _top3_row(row_ref, lane_iota):
  bvec = _block_maxima(row_ref, lane_iota)
  block_ids = _top3_block_ids(bvec, lane_iota)
  results = []
  for p in range(3):
    i = _argmax_blocks(row_ref, lane_iota, block_ids)
    results.append(i)
    if p < 2:
      c1 = i // LANES
      l1 = i - c1 * LANES
      chunk = row_ref[pl.ds(c1 * LANES, LANES)]
      row_ref[pl.ds(c1 * LANES, LANES)] = jnp.where(
          lane_iota == l1, _NEG_INF, chunk)
  return results


def _sc_kernel(x_hbm, out_hbm, buf0, buf1, outbuf, sem0, sem1):
  wid = lax.axis_index("c") * 16 + lax.axis_index("s")
  r0 = wid * ROWS_PER_WORKER
  lane_iota = lax.broadcasted_iota(jnp.int32, (LANES,), 0)

  cp0 = pltpu.async_copy(x_hbm.at[r0], buf0, sem0)
  cp1 = pltpu.async_copy(x_hbm.at[r0 + 1], buf1, sem1)

  cp0.wait()
  a1, a2, a3 = _top3_row(buf0, lane_iota)
  cp1.wait()
  b1, b2, b3 = _top3_row(buf1, lane_iota)

  vals = [a1, a2, a3, b1, b2, b3]
  res = jnp.zeros((LANES,), jnp.float32)
  for lane, v in enumerate(vals):
    res = jnp.where(lane_iota == lane, v.astype(jnp.float32), res)
  outbuf[...] = res
  pltpu.sync_copy(outbuf, out_hbm.at[wid])


@jax.jit
def kernel(x):
  mesh = plsc.VectorSubcoreMesh(core_axis_name="c", subcore_axis_name="s")
  k = pl.kernel(
      _sc_kernel,
      out_type=jax.ShapeDtypeStruct((NWORKERS, LANES), jnp.float32),
      mesh=mesh,
      compiler_params=pltpu.CompilerParams(needs_layout_passes=False),
      scratch_types=[
          pltpu.VMEM((COLS,), jnp.float32),
          pltpu.VMEM((COLS,), jnp.float32),
          pltpu.VMEM((LANES,), jnp.float32),
          pltpu.SemaphoreType.DMA,
          pltpu.SemaphoreType.DMA,
      ],
  )
  staged = k(x)
  return staged[:, :6].reshape(ROWS, 3)


# speedup vs baseline: 2.6389x; 1.0001x over previous
import jax
import jax.numpy as jnp
from jax import lax
from jax.experimental import pallas as pl
from jax.experimental.pallas import tpu as pltpu
from jax.experimental.pallas import tpu_sc as plsc

ROWS = 64
COLS = 32768
LANES = 16
NWORKERS = 32
ROWS_PER_WORKER = ROWS // NWORKERS

NBLK = 16
BCHUNKS = COLS // (NBLK * LANES)

_NEG_INF = float("-inf")
_BIG_I32 = 2**30


def _block_maxima(row_ref, lane_iota):
  ninf = jnp.full((LANES,), _NEG_INF, jnp.float32)

  def blk_body(j, bvec):
    base = j * (BCHUNKS * LANES)

    def body(c, accs):
      a0, a1, a2, a3 = accs
      o = base + c * (16 * LANES)
      for u in range(0, 16, 4):
        a0 = jnp.maximum(a0, row_ref[pl.ds(o + (u + 0) * LANES, LANES)])
        a1 = jnp.maximum(a1, row_ref[pl.ds(o + (u + 1) * LANES, LANES)])
        a2 = jnp.maximum(a2, row_ref[pl.ds(o + (u + 2) * LANES, LANES)])
        a3 = jnp.maximum(a3, row_ref[pl.ds(o + (u + 3) * LANES, LANES)])
      return a0, a1, a2, a3

    a0, a1, a2, a3 = lax.fori_loop(0, BCHUNKS // 16, body,
                                   (ninf, ninf, ninf, ninf))
    acc = jnp.maximum(jnp.maximum(a0, a1), jnp.maximum(a2, a3))
    bm = jnp.max(acc)
    return jnp.where(lane_iota == j, bm, bvec)

  return lax.fori_loop(0, NBLK, blk_body, ninf)


def _top3_block_ids(bvec, lane_iota):
  ids = []
  b = bvec
  for _ in range(3):
    m = jnp.max(b)
    j = jnp.min(jnp.where(b == m, lane_iota, _BIG_I32))
    ids.append(j)
    b = jnp.where(lane_iota == j, _NEG_INF, b)
  a, bb, c = ids
  lo = jnp.minimum(jnp.minimum(a, bb), c)
  hi = jnp.maximum(jnp.maximum(a, bb), c)
  mid = a + bb + c - lo - hi
  return lo, mid, hi


def _argmax_blocks(row_ref, lane_iota, block_ids):
  ninf = jnp.full((LANES,), _NEG_INF, jnp.float32)
  zero = jnp.zeros((LANES,), jnp.int32)
  carry = (ninf, zero, ninf, zero, ninf, zero, ninf, zero)

  for j in block_ids:
    cbase = j * BCHUNKS

    def body(c, accs, cbase=cbase):
      b0, c0, b1, c1, b2, c2, b3, c3 = accs
      cc = cbase + c * 8
      o = cc * LANES
      bs = [b0, b1, b2, b3]
      cs = [c0, c1, c2, c3]
      for u in range(8):
        k = u % 4
        v = row_ref[pl.ds(o + u * LANES, LANES)]
        m = v > bs[k]
        bs[k] = jnp.where(m, v, bs[k])
        cs[k] = jnp.where(m, cc + u, cs[k])
      return (bs[0], cs[0], bs[1], cs[1], bs[2], cs[2], bs[3], cs[3])

    carry = lax.fori_loop(0, BCHUNKS // 8, body, carry)

  def merge(bv_a, cv_a, bv_b, cv_b):
    take = (bv_b > bv_a) | ((bv_b == bv_a) & (cv_b < cv_a))
    return jnp.where(take, bv_b, bv_a), jnp.where(take, cv_b, cv_a)

  b0, c0, b1, c1, b2, c2, b3, c3 = carry
  ba, ca = merge(b0, c0, b1, c1)
  bb, cb = merge(b2, c2, b3, c3)
  best, bestc = merge(ba, ca, bb, cb)

  idx = bestc * LANES + lane_iota
  maxv = jnp.max(best)
  return jnp.min(jnp.where(best == maxv, idx, _BIG_I32))


def _top3_row(row_ref, lane_iota):
  bvec = _block_maxima(row_ref, lane_iota)
  block_ids = _top3_block_ids(bvec, lane_iota)
  results = []
  for p in range(3):
    i = _argmax_blocks(row_ref, lane_iota, block_ids)
    results.append(i)
    if p < 2:
      c1 = i // LANES
      l1 = i - c1 * LANES
      chunk = row_ref[pl.ds(c1 * LANES, LANES)]
      row_ref[pl.ds(c1 * LANES, LANES)] = jnp.where(
          lane_iota == l1, _NEG_INF, chunk)
  return results


def _sc_kernel(x_hbm, out_hbm, buf0, buf1, outbuf, sem0, sem1):
  wid = lax.axis_index("c") * 16 + lax.axis_index("s")
  r0 = wid * ROWS_PER_WORKER
  lane_iota = lax.broadcasted_iota(jnp.int32, (LANES,), 0)

  cp0 = pltpu.async_copy(x_hbm.at[r0], buf0, sem0)
  cp1 = pltpu.async_copy(x_hbm.at[r0 + 1], buf1, sem1)

  cp0.wait()
  a1, a2, a3 = _top3_row(buf0, lane_iota)
  cp1.wait()
  b1, b2, b3 = _top3_row(buf1, lane_iota)

  vals = [a1, a2, a3, b1, b2, b3]
  res = jnp.zeros((LANES,), jnp.float32)
  for lane, v in enumerate(vals):
    res = jnp.where(lane_iota == lane, v.astype(jnp.float32), res)
  outbuf[...] = res
  pltpu.sync_copy(outbuf, out_hbm.at[wid])


@jax.jit
def kernel(x):
  mesh = plsc.VectorSubcoreMesh(core_axis_name="c", subcore_axis_name="s")
  k = pl.kernel(
      _sc_kernel,
      out_type=jax.ShapeDtypeStruct((NWORKERS, LANES), jnp.float32),
      mesh=mesh,
      compiler_params=pltpu.CompilerParams(needs_layout_passes=False, skip_device_barrier=True),
      scratch_types=[
          pltpu.VMEM((COLS,), jnp.float32),
          pltpu.VMEM((COLS,), jnp.float32),
          pltpu.VMEM((LANES,), jnp.float32),
          pltpu.SemaphoreType.DMA,
          pltpu.SemaphoreType.DMA,
      ],
  )
  staged = k(x)
  return staged[:, :6].reshape(ROWS, 3)
